# Initial kernel scaffold; baseline (speedup 1.0000x reference)
import jax
import jax.numpy as jnp
from jax.experimental import pallas as pl


def kernel(x, edge_index, batch, W1, as1, ad1, b1, W2, as2, ad2, b2, W3, as3, ad3, b3, W4, as4, ad4, b4, Wl, bl):
    raise NotImplementedError("write your pallas kernel here")


# speedup vs baseline: 1.0421x; 1.0421x over previous
import jax
import jax.numpy as jnp
from jax.experimental import pallas as pl


def _split3(a):
    a0 = a.astype(jnp.bfloat16).astype(jnp.float32)
    r = a - a0
    a1 = r.astype(jnp.bfloat16).astype(jnp.float32)
    a2 = r - a1
    return a0, a1, a2


def _f32_dot(a, b):
    a0, a1, a2 = _split3(a)
    b0, b1, b2 = _split3(b)
    d = lambda u, v: jnp.dot(u, v, preferred_element_type=jnp.float32)
    t2 = d(a0, b2) + d(a1, b1) + d(a2, b0)
    t1 = d(a0, b1) + d(a1, b0)
    return (t2 + t1) + d(a0, b0)


def _proj_attn_kernel(x_ref, w_ref, asrc_ref, adst_ref, h_ref, s_ref, d_ref):
    h = _f32_dot(x_ref[...], w_ref[...])
    h_ref[...] = h
    s_ref[...] = _f32_dot(h, asrc_ref[...])
    d_ref[...] = _f32_dot(h, adst_ref[...])


def _pack_attn(a, out_dim):
    heads, ch = a.shape
    A = jnp.zeros((out_dim, heads), dtype=a.dtype)
    rows = jnp.arange(out_dim)
    A = A.at[rows, rows // ch].set(a.reshape(-1))
    return A


def _proj_attn(x, W, a_src, a_dst):
    n, in_dim = x.shape
    out_dim = W.shape[1]
    heads = a_src.shape[0]
    Asrc = _pack_attn(a_src, out_dim)
    Adst = _pack_attn(a_dst, out_dim)
    bn = 1000
    grid = n // bn
    h, s, d = pl.pallas_call(
        _proj_attn_kernel,
        grid=(grid,),
        in_specs=[
            pl.BlockSpec((bn, in_dim), lambda i: (i, 0)),
            pl.BlockSpec((in_dim, out_dim), lambda i: (0, 0)),
            pl.BlockSpec((out_dim, heads), lambda i: (0, 0)),
            pl.BlockSpec((out_dim, heads), lambda i: (0, 0)),
        ],
        out_specs=[
            pl.BlockSpec((bn, out_dim), lambda i: (i, 0)),
            pl.BlockSpec((bn, heads), lambda i: (i, 0)),
            pl.BlockSpec((bn, heads), lambda i: (i, 0)),
        ],
        out_shape=[
            jax.ShapeDtypeStruct((n, out_dim), jnp.float32),
            jax.ShapeDtypeStruct((n, heads), jnp.float32),
            jax.ShapeDtypeStruct((n, heads), jnp.float32),
        ],
    )(x, W, Asrc, Adst)
    return h, s, d


def _gat_layer(x, W, a_src, a_dst, b, src, dst, heads, ch):
    n = x.shape[0]
    h2, asrc, adst = _proj_attn(x, W, a_src, a_dst)
    h = h2.reshape(n, heads, ch)
    e = jax.nn.leaky_relu(asrc[src] + adst[dst], negative_slope=0.2)
    emax = jax.ops.segment_max(e, dst, num_segments=n)
    emax = jnp.where(jnp.isfinite(emax), emax, 0.0)
    ez = jnp.exp(e - emax[dst])
    den = jax.ops.segment_sum(ez, dst, num_segments=n)
    alpha = ez / (den[dst] + 1e-16)
    out = jax.ops.segment_sum(h[src] * alpha[:, :, None], dst, num_segments=n)
    return out.reshape(n, heads * ch) + b


def kernel(x, edge_index, batch, W1, as1, ad1, b1, W2, as2, ad2, b2, W3, as3, ad3, b3, W4, as4, ad4, b4, Wl, bl):
    n = x.shape[0]
    ng = 64
    loop = jnp.arange(n, dtype=edge_index.dtype)
    src = jnp.concatenate([edge_index[0], loop])
    dst = jnp.concatenate([edge_index[1], loop])
    h = _gat_layer(x, W1, as1, ad1, b1, src, dst, 8, 64)
    h = jax.nn.elu(h)
    h = _gat_layer(h, W2, as2, ad2, b2, src, dst, 1, 512)
    h = jax.nn.elu(h)
    h = _gat_layer(h, W3, as3, ad3, b3, src, dst, 1, 512)
    h = jax.nn.elu(h)
    hg = _gat_layer(h, W4, as4, ad4, b4, src, dst, 1, 64)
    counts = jnp.bincount(batch, length=ng)
    max_count = jnp.maximum(jnp.max(counts), 1).astype(jnp.float32)
    gsum = jax.ops.segment_sum(hg, batch, num_segments=ng)
    x_new = gsum / max_count
    return x_new @ Wl + bl
